# baseline (speedup 1.0000x reference)
import functools

import jax
import jax.numpy as jnp
from jax import lax
from jax.experimental import pallas as pl
from jax.experimental.pallas import tpu as pltpu
from jax.experimental.pallas import tpu_sc as plsc

_L = 16
_NS = 16
_ROWS = 16384
_CHUNK = _ROWS // _NS
_ITERS = _CHUNK // _L


def _fprate_sc(out_hbm, tgt_hbm, res_hbm, part_hbm,
               out_v, tgt_v, part_v, res_v, loc):
    cid = lax.axis_index("c")
    sid = lax.axis_index("s")

    @pl.when(cid == 0)
    def _core0():
        base = sid * _CHUNK
        pltpu.sync_copy(out_hbm.at[pl.ds(2 * base, 2 * _CHUNK)], out_v)
        pltpu.sync_copy(tgt_hbm.at[pl.ds(base, _CHUNK)], tgt_v)

        lanes = lax.broadcasted_iota(jnp.int32, (_L,), 0)
        one = jnp.ones((_L,), jnp.int32)
        zero = jnp.zeros((_L,), jnp.int32)

        acc_fp = zero
        acc_n0 = zero
        for j in range(_ITERS):
            even = 2 * _L * j + 2 * lanes
            c0 = plsc.load_gather(out_v, [even])
            c1 = plsc.load_gather(out_v, [even + 1])
            t0 = tgt_v[pl.ds(j * _L, _L)] == 0
            acc_fp = acc_fp + jnp.where((c1 > c0) & t0, one, zero)
            acc_n0 = acc_n0 + jnp.where(t0, one, zero)

        part_v[0, :] = acc_fp.astype(jnp.float32)
        part_v[1, :] = acc_n0.astype(jnp.float32)
        pltpu.sync_copy(part_v, part_hbm.at[sid])
        plsc.subcore_barrier()

        @pl.when(sid == 0)
        def _finalize():
            pltpu.sync_copy(part_hbm, loc)
            fp_vec = loc[0, 0]
            n0_vec = loc[0, 1]
            for s in range(1, _NS):
                fp_vec = fp_vec + loc[s, 0]
                n0_vec = n0_vec + loc[s, 1]
            fp = jnp.full((_L,), jnp.sum(fp_vec), jnp.float32)
            n0 = jnp.full((_L,), jnp.sum(n0_vec), jnp.float32)
            res_v[...] = fp / (n0 + 1e-10)
            pltpu.sync_copy(res_v, res_hbm)


def kernel(output, target):
    target = target.astype(jnp.int32)
    run = functools.partial(
        pl.kernel,
        mesh=plsc.VectorSubcoreMesh(core_axis_name="c", subcore_axis_name="s"),
        compiler_params=pltpu.CompilerParams(needs_layout_passes=False),
        out_type=(
            jax.ShapeDtypeStruct((_L,), jnp.float32),
            jax.ShapeDtypeStruct((_NS, 2, _L), jnp.float32),
        ),
        scratch_types=[
            pltpu.VMEM((2 * _CHUNK,), jnp.float32),
            pltpu.VMEM((_CHUNK,), jnp.int32),
            pltpu.VMEM((2, _L), jnp.float32),
            pltpu.VMEM((_L,), jnp.float32),
            pltpu.VMEM((_NS, 2, _L), jnp.float32),

        ],
    )(_fprate_sc)
    res, _ = run(output.reshape(-1), target)
    return res[0]
